# Initial kernel scaffold; baseline (speedup 1.0000x reference)
import functools

import jax
import jax.numpy as jnp
from jax import lax
from jax.experimental import pallas as pl
from jax.experimental.pallas import tpu as pltpu
from jax.experimental.pallas import tpu_sc as plsc

N_ = 10000
E_ = 320000
D_ = 128
H_ = 4
F_ = 32
NP_ = 10240
ROW = 144
B_ = 80
NW = 32
EPT = E_ // NW
NCH = EPT // B_
RPT = NP_ // 16
BN = 400


def _pro_body(feat_ref, wt_ref, m_ref, h_ref, elr_ref, mx_ref):
    hb = jnp.dot(feat_ref[...], wt_ref[...],
                 preferred_element_type=jnp.float32,
                 precision=lax.Precision.HIGHEST)
    h_ref[...] = hb
    elr = jnp.dot(hb, m_ref[...], preferred_element_type=jnp.float32,
                  precision=lax.Precision.HIGHEST)
    elr_ref[...] = elr
    bm = jnp.max(elr, axis=0, keepdims=True)

    @pl.when(pl.program_id(0) == 0)
    def _():
        mx_ref[...] = bm

    @pl.when(pl.program_id(0) != 0)
    def _():
        mx_ref[...] = jnp.maximum(mx_ref[...], bm)


_prologue = pl.pallas_call(
    _pro_body,
    grid=(N_ // BN,),
    in_specs=[pl.BlockSpec((BN, D_), lambda i: (i, 0)),
              pl.BlockSpec((D_, D_), lambda i: (0, 0)),
              pl.BlockSpec((D_, 8), lambda i: (0, 0))],
    out_specs=[pl.BlockSpec((BN, D_), lambda i: (i, 0)),
               pl.BlockSpec((BN, 8), lambda i: (i, 0)),
               pl.BlockSpec((1, 8), lambda i: (0, 0))],
    out_shape=[jax.ShapeDtypeStruct((N_, D_), jnp.float32),
               jax.ShapeDtypeStruct((N_, 8), jnp.float32),
               jax.ShapeDtypeStruct((1, 8), jnp.float32)],
)


def _sc_body(src_hbm, dst_hbm, elr_hbm, mx_hbm, h_hbm, out_hbm,
             elr_t, sidx, didx, hbuf, msgbuf, eebuf, mxbuf, zbuf, acc, sem):
    cid = lax.axis_index("c")
    sid = lax.axis_index("s")
    wid = cid * 16 + sid

    pltpu.sync_copy(elr_hbm, elr_t)
    pltpu.sync_copy(mx_hbm, mxbuf)

    zv = jnp.zeros((16,), jnp.float32)

    def zrow(r, carry):
        for c9 in range(ROW // 16):
            zbuf[r, pl.ds(c9 * 16, 16)] = zv
        return carry

    lax.fori_loop(0, 64, zrow, 0)
    eebuf[pl.ds(64, 16)] = zv
    for k in range(RPT // 64):
        pltpu.sync_copy(zbuf, acc.at[pl.ds(sid * RPT + k * 64, 64), :])
    plsc.subcore_barrier()

    mxs = [mxbuf[0], mxbuf[1], mxbuf[2], mxbuf[3]]
    ebase = wid * EPT
    tailidx = jnp.array([0, 16, 32, 48] + [64] * 12, jnp.int32)

    def chunk(i, carry):
        off = ebase + i * B_
        pltpu.sync_copy(src_hbm.at[pl.ds(off, B_)], sidx)
        pltpu.sync_copy(dst_hbm.at[pl.ds(off, B_)], didx)
        pltpu.async_copy(h_hbm.at[sidx], hbuf, sem).wait()
        for g in range(B_ // 16):
            s16 = sidx[pl.ds(g * 16, 16)]
            d16 = didx[pl.ds(g * 16, 16)]
            s8 = s16 * 8
            d8 = d16 * 8
            for hh in range(4):
                elh = plsc.load_gather(elr_t, [s8 + hh])
                erh = plsc.load_gather(elr_t, [d8 + (4 + hh)])
                x = elh + erh
                e = jnp.maximum(x, 0.2 * x)
                y = mxs[hh] + erh
                cc = jnp.maximum(y, 0.2 * y)
                eebuf[pl.ds(hh * 16, 16)] = jnp.exp(e - cc)
            for j in range(16):
                row = g * 16 + j
                tail = plsc.load_gather(eebuf, [tailidx + j])
                msgbuf[row, pl.ds(128, 16)] = tail
                for hh in range(4):
                    s = eebuf[hh * 16 + j]
                    c0 = hh * 32
                    msgbuf[row, pl.ds(c0, 16)] = hbuf[row, pl.ds(c0, 16)] * s
                    msgbuf[row, pl.ds(c0 + 16, 16)] = (
                        hbuf[row, pl.ds(c0 + 16, 16)] * s)
        pltpu.sync_copy(msgbuf, acc.at[didx], add=True)
        return carry

    lax.fori_loop(0, NCH, chunk, 0)
    plsc.subcore_barrier()
    for k in range(RPT // 64):
        r0 = sid * RPT + k * 64
        pltpu.sync_copy(acc.at[pl.ds(r0, 64), :],
                        out_hbm.at[cid, pl.ds(r0, 64), :])


_sc_kernel = functools.partial(
    pl.kernel,
    mesh=plsc.VectorSubcoreMesh(core_axis_name="c", subcore_axis_name="s"),
    out_type=jax.ShapeDtypeStruct((2, NP_, ROW), jnp.float32),
    scratch_types=[
        pltpu.VMEM((8 * N_,), jnp.float32),
        pltpu.VMEM((B_,), jnp.int32),
        pltpu.VMEM((B_,), jnp.int32),
        pltpu.VMEM((B_, D_), jnp.float32),
        pltpu.VMEM((B_, ROW), jnp.float32),
        pltpu.VMEM((80,), jnp.float32),
        pltpu.VMEM((16,), jnp.float32),
        pltpu.VMEM((64, ROW), jnp.float32),
        pltpu.VMEM_SHARED((NP_, ROW), jnp.float32),
        pltpu.SemaphoreType.DMA,
    ],
)(_sc_body)


def _fin_body(p_ref, r_ref, b_ref, o_ref):
    blk = p_ref[0] + p_ref[1]
    msg = blk[:, :128]
    den = blk[:, 128:132]
    dinv = 1.0 / jnp.maximum(den, 1e-30)
    o_ref[...] = msg * jnp.dot(dinv, r_ref[...],
                               preferred_element_type=jnp.float32,
                               precision=lax.Precision.HIGHEST) + b_ref[...]


_final = pl.pallas_call(
    _fin_body,
    grid=(N_ // BN,),
    in_specs=[pl.BlockSpec((2, BN, ROW), lambda i: (0, i, 0)),
              pl.BlockSpec((H_, D_), lambda i: (0, 0)),
              pl.BlockSpec((1, D_), lambda i: (0, 0))],
    out_specs=pl.BlockSpec((BN, D_), lambda i: (i, 0)),
    out_shape=jax.ShapeDtypeStruct((N_, D_), jnp.float32),
)


def kernel(feat, edge_index, W, attn_l, attn_r, bias):
    al = attn_l.reshape(H_ * F_)
    ar = attn_r.reshape(H_ * F_)
    hf = jnp.arange(D_)
    headcol = hf // F_
    M = jnp.zeros((D_, 8), jnp.float32)
    M = M.at[hf, headcol].set(al)
    M = M.at[hf, 4 + headcol].set(ar)
    R = jnp.zeros((H_, D_), jnp.float32).at[headcol, hf].set(1.0)

    h_, elr_, mx_ = _prologue(feat, W.T, M)
    src = edge_index[0]
    dst = edge_index[1]
    part = _sc_kernel(src, dst, elr_.reshape(8 * N_),
                      jnp.pad(mx_.reshape(8), (0, 8)), h_)
    out = _final(part, R, bias.reshape(1, D_))
    return out.reshape(N_, H_, F_)


# speedup vs baseline: 24.5797x; 24.5797x over previous
import functools

import jax
import jax.numpy as jnp
from jax import lax
from jax.experimental import pallas as pl
from jax.experimental.pallas import tpu as pltpu
from jax.experimental.pallas import tpu_sc as plsc

N_ = 10000
E_ = 320000
D_ = 128
H_ = 4
F_ = 32
NP_ = 10240
DR_ = 320
B_ = 80
NG_ = B_ // 16
NW = 32
EPT = E_ // NW
NCH = EPT // B_
BN = 512

_SC_PARAMS = pltpu.CompilerParams(needs_layout_passes=False)
_MESH = plsc.VectorSubcoreMesh(core_axis_name="c", subcore_axis_name="s")


def _pro_body(feat_ref, wt_ref, m_ref, h_ref, elr_ref, mx_ref):
    hb = jnp.dot(feat_ref[...], wt_ref[...],
                 preferred_element_type=jnp.float32,
                 precision=lax.Precision.HIGHEST)
    h_ref[...] = hb
    elr = jnp.dot(hb, m_ref[...], preferred_element_type=jnp.float32,
                  precision=lax.Precision.HIGHEST)
    elr_ref[...] = elr
    bm = jnp.max(elr, axis=0, keepdims=True)

    @pl.when(pl.program_id(0) == 0)
    def _():
        mx_ref[...] = bm

    @pl.when(pl.program_id(0) != 0)
    def _():
        mx_ref[...] = jnp.maximum(mx_ref[...], bm)


_prologue = pl.pallas_call(
    _pro_body,
    grid=(N_ // 400,),
    in_specs=[pl.BlockSpec((400, D_), lambda i: (i, 0)),
              pl.BlockSpec((D_, D_), lambda i: (0, 0)),
              pl.BlockSpec((D_, 8), lambda i: (0, 0))],
    out_specs=[pl.BlockSpec((400, D_), lambda i: (i, 0)),
               pl.BlockSpec((400, 8), lambda i: (i, 0)),
               pl.BlockSpec((1, 8), lambda i: (0, 0))],
    out_shape=[jax.ShapeDtypeStruct((N_, D_), jnp.float32),
               jax.ShapeDtypeStruct((N_, 8), jnp.float32),
               jax.ShapeDtypeStruct((1, 8), jnp.float32)],
)


def _pa_body(src_hbm, dst_hbm, elr_hbm, mx_hbm, ee_hbm, den_hbm,
             elr_t, den_t, sidx, didx, stg, mxbuf, zbuf,
             ridx0, ridx1, ridx2, ridx3, ridx4, acc_den, sem):
    ridxs = [ridx0, ridx1, ridx2, ridx3, ridx4]
    cid = lax.axis_index("c")
    sid = lax.axis_index("s")
    wid = cid * 16 + sid

    pltpu.sync_copy(elr_hbm, elr_t)
    pltpu.sync_copy(mx_hbm, mxbuf)

    zv = jnp.zeros((16,), jnp.float32)
    iota = lax.iota(jnp.int32, 16)

    def zrow(r, carry):
        for q in range(8):
            zbuf[r, pl.ds(q * 16, 16)] = zv
        return carry

    lax.fori_loop(0, 16, zrow, 0)

    def zden(r, carry):
        for q in range(8):
            den_t[r, pl.ds(q * 16, 16)] = zv
        return carry

    lax.fori_loop(0, DR_, zden, 0)
    for r in range(5):
        for q in range(4):
            ridxs[r][pl.ds(q * 16, 16)] = iota + (r * 64 + q * 16)
    @pl.when(sid < 10)
    def _():
        pltpu.sync_copy(zbuf, acc_den.at[pl.ds(sid * 32, 16), :])
        pltpu.sync_copy(zbuf, acc_den.at[pl.ds(sid * 32 + 16, 16), :])

    plsc.subcore_barrier()

    mxv = mxbuf[pl.ds(0, 16)]
    mxs = [mxv[0], mxv[1], mxv[2], mxv[3]]
    ebase = wid * EPT

    def chunk(i, carry):
        off = ebase + i * B_
        pltpu.sync_copy(src_hbm.at[pl.ds(off, B_)], sidx)
        pltpu.sync_copy(dst_hbm.at[pl.ds(off, B_)], didx)
        for g in range(NG_):
            s16 = sidx[pl.ds(g * 16, 16)]
            d16 = didx[pl.ds(g * 16, 16)]
            s8 = s16 * 8
            d8 = d16 * 8
            d4 = d16 * 4
            stg_base = g * 64 + iota * 4
            for hh in range(4):
                elh = plsc.load_gather(elr_t, [s8 + hh])
                erh = plsc.load_gather(elr_t, [d8 + (4 + hh)])
                x = elh + erh
                e = jnp.maximum(x, 0.2 * x)
                y = mxs[hh] + erh
                cc = jnp.maximum(y, 0.2 * y)
                ee = jnp.exp(e - cc)
                plsc.store_scatter(stg, [stg_base + hh], ee)
                idx = d4 + hh
                plsc.addupdate_scatter(den_t, [idx >> 7, idx & 127], ee)
        pltpu.sync_copy(stg, ee_hbm.at[pl.ds(off * 4, 4 * B_)])
        return carry

    lax.fori_loop(0, NCH, chunk, 0)

    for r in range(5):
        pltpu.sync_copy(den_t.at[pl.ds(r * 64, 64), :],
                        acc_den.at[ridxs[r]], add=True)
    plsc.subcore_barrier()

    @pl.when(sid < 10)
    def _():
        pltpu.sync_copy(acc_den.at[pl.ds(sid * 32, 32), :],
                        den_hbm.at[cid, pl.ds(sid * 32, 32), :])


_pass_a = functools.partial(
    pl.kernel,
    mesh=_MESH,
    compiler_params=_SC_PARAMS,
    out_type=[jax.ShapeDtypeStruct((4 * E_,), jnp.float32),
              jax.ShapeDtypeStruct((2, DR_, D_), jnp.float32)],
    scratch_types=[
        pltpu.VMEM((8 * N_,), jnp.float32),
        pltpu.VMEM((DR_, D_), jnp.float32),
        pltpu.VMEM((B_,), jnp.int32),
        pltpu.VMEM((B_,), jnp.int32),
        pltpu.VMEM((4 * B_,), jnp.float32),
        pltpu.VMEM((16,), jnp.float32),
        pltpu.VMEM((16, D_), jnp.float32),
        pltpu.VMEM((64,), jnp.int32),
        pltpu.VMEM((64,), jnp.int32),
        pltpu.VMEM((64,), jnp.int32),
        pltpu.VMEM((64,), jnp.int32),
        pltpu.VMEM((64,), jnp.int32),
        pltpu.VMEM_SHARED((DR_, D_), jnp.float32),
        pltpu.SemaphoreType.DMA,
    ],
)(_pa_body)


def _mid_body(p_ref, o_ref):
    s = p_ref[0] + p_ref[1]
    o_ref[...] = 1.0 / jnp.maximum(s, 1e-30)


_mid = pl.pallas_call(
    _mid_body,
    out_shape=jax.ShapeDtypeStruct((DR_, D_), jnp.float32),
)


HNP = NP_ // 2
EPTB = E_ // 16
NCHB = EPTB // B_


def _pb_body(src_hbm, dst_hbm, ee_hbm, invd_hbm, h_hbm, out_hbm,
             invd_t, sidx, didx, didx2, stg, hbuf, zbuf, acc, sem):
    cid = lax.axis_index("c")
    sid = lax.axis_index("s")

    pltpu.sync_copy(invd_hbm, invd_t)

    zv = jnp.zeros((16,), jnp.float32)
    iota = lax.iota(jnp.int32, 16)

    def zrow(r, carry):
        for q in range(8):
            zbuf[r, pl.ds(q * 16, 16)] = zv
        return carry

    lax.fori_loop(0, 64, zrow, 0)
    for k in range(5):
        pltpu.sync_copy(zbuf, acc.at[pl.ds(sid * 320 + k * 64, 64), :])

    @pl.when(sid == 15)
    def _():
        pltpu.sync_copy(zbuf.at[pl.ds(0, 8), :], acc.at[pl.ds(HNP, 8), :])

    plsc.subcore_barrier()

    lo = cid * HNP
    ebase = sid * EPTB

    def chunk(i, carry):
        off = ebase + i * B_
        pltpu.sync_copy(src_hbm.at[pl.ds(off, B_)], sidx)
        pltpu.sync_copy(dst_hbm.at[pl.ds(off, B_)], didx)
        pltpu.sync_copy(ee_hbm.at[pl.ds(off * 4, 4 * B_)], stg)
        pltpu.async_copy(h_hbm.at[sidx], hbuf, sem).wait()
        for g in range(NG_):
            d16 = didx[pl.ds(g * 16, 16)]
            dl = d16 - lo
            inr = (dl >= 0) & (dl < HNP)
            didx2[pl.ds(g * 16, 16)] = jnp.where(inr, dl, HNP)
            d4 = d16 * 4
            stg_base = g * 64 + iota * 4
            avecs = []
            for hh in range(4):
                eev = plsc.load_gather(stg, [stg_base + hh])
                inv = plsc.load_gather(invd_t, [d4 + hh])
                avecs.append(eev * inv)
            for j in range(16):
                row = g * 16 + j
                for hh in range(4):
                    s = avecs[hh][j]
                    c0 = hh * 32
                    hbuf[row, pl.ds(c0, 16)] = hbuf[row, pl.ds(c0, 16)] * s
                    hbuf[row, pl.ds(c0 + 16, 16)] = (
                        hbuf[row, pl.ds(c0 + 16, 16)] * s)
        pltpu.sync_copy(hbuf, acc.at[didx2], add=True)
        return carry

    lax.fori_loop(0, NCHB, chunk, 0)
    plsc.subcore_barrier()
    for k in range(5):
        r0 = sid * 320 + k * 64
        pltpu.sync_copy(acc.at[pl.ds(r0, 64), :],
                        out_hbm.at[cid, pl.ds(r0, 64), :])


_pass_b = functools.partial(
    pl.kernel,
    mesh=_MESH,
    compiler_params=_SC_PARAMS,
    out_type=jax.ShapeDtypeStruct((2, HNP, D_), jnp.float32),
    scratch_types=[
        pltpu.VMEM((4 * NP_,), jnp.float32),
        pltpu.VMEM((B_,), jnp.int32),
        pltpu.VMEM((B_,), jnp.int32),
        pltpu.VMEM((B_,), jnp.int32),
        pltpu.VMEM((4 * B_,), jnp.float32),
        pltpu.VMEM((B_, D_), jnp.float32),
        pltpu.VMEM((64, D_), jnp.float32),
        pltpu.VMEM_SHARED((HNP + 8, D_), jnp.float32),
        pltpu.SemaphoreType.DMA,
    ],
)(_pb_body)


def _fin_body(p_ref, b_ref, o_ref):
    o_ref[...] = p_ref[0] + b_ref[...]


_final = pl.pallas_call(
    _fin_body,
    grid=(NP_ // BN,),
    in_specs=[pl.BlockSpec((1, BN, D_), lambda i: (i // (HNP // BN),
                                                   i % (HNP // BN), 0)),
              pl.BlockSpec((1, D_), lambda i: (0, 0))],
    out_specs=pl.BlockSpec((BN, D_), lambda i: (i, 0)),
    out_shape=jax.ShapeDtypeStruct((NP_, D_), jnp.float32),
)


def kernel(feat, edge_index, W, attn_l, attn_r, bias):
    al = attn_l.reshape(H_ * F_)
    ar = attn_r.reshape(H_ * F_)
    hf = jnp.arange(D_)
    headcol = hf // F_
    M = jnp.zeros((D_, 8), jnp.float32)
    M = M.at[hf, headcol].set(al)
    M = M.at[hf, 4 + headcol].set(ar)

    h_, elr_, mx_ = _prologue(feat, W.T, M)
    src = edge_index[0]
    dst = edge_index[1]
    ee, den_part = _pass_a(src, dst, elr_.reshape(8 * N_),
                           jnp.pad(mx_.reshape(8), (0, 8)))
    invd = _mid(den_part)
    msg_part = _pass_b(src, dst, ee, invd.reshape(4 * NP_), h_)
    out = _final(msg_part, bias.reshape(1, D_))
    return out[:N_].reshape(N_, H_, F_)


# speedup vs baseline: 34.2534x; 1.3936x over previous
import functools

import jax
import jax.numpy as jnp
from jax import lax
from jax.experimental import pallas as pl
from jax.experimental.pallas import tpu as pltpu
from jax.experimental.pallas import tpu_sc as plsc

N_ = 10000
E_ = 320000
D_ = 128
H_ = 4
F_ = 32
NP_ = 10240
DR_ = 320
B_ = 400
NG_ = B_ // 16
NW = 32
EPT = E_ // NW
NCH = EPT // B_
BB_ = 160
NGB_ = BB_ // 16
BN = 512

_SC_PARAMS = pltpu.CompilerParams(needs_layout_passes=False)
_MESH = plsc.VectorSubcoreMesh(core_axis_name="c", subcore_axis_name="s")


def _pro_body(feat_ref, wt_ref, m_ref, h_ref, elr_ref, mx_ref):
    hb = jnp.dot(feat_ref[...], wt_ref[...],
                 preferred_element_type=jnp.float32,
                 precision=lax.Precision.HIGHEST)
    h_ref[...] = hb
    elr = jnp.dot(hb, m_ref[...], preferred_element_type=jnp.float32,
                  precision=lax.Precision.HIGHEST)
    elr_ref[...] = elr
    bm = jnp.max(elr, axis=0, keepdims=True)

    @pl.when(pl.program_id(0) == 0)
    def _():
        mx_ref[...] = bm

    @pl.when(pl.program_id(0) != 0)
    def _():
        mx_ref[...] = jnp.maximum(mx_ref[...], bm)


_prologue = pl.pallas_call(
    _pro_body,
    grid=(N_ // 400,),
    in_specs=[pl.BlockSpec((400, D_), lambda i: (i, 0)),
              pl.BlockSpec((D_, D_), lambda i: (0, 0)),
              pl.BlockSpec((D_, 8), lambda i: (0, 0))],
    out_specs=[pl.BlockSpec((400, D_), lambda i: (i, 0)),
               pl.BlockSpec((400, 8), lambda i: (i, 0)),
               pl.BlockSpec((1, 8), lambda i: (0, 0))],
    out_shape=[jax.ShapeDtypeStruct((N_, D_), jnp.float32),
               jax.ShapeDtypeStruct((N_, 8), jnp.float32),
               jax.ShapeDtypeStruct((1, 8), jnp.float32)],
)


def _pa_body(src_hbm, dst_hbm, elr_hbm, mx_hbm, ee_hbm, den_hbm,
             elr_t, den_t, sidx, didx, stg, mxbuf, zbuf,
             ridx0, ridx1, ridx2, ridx3, ridx4, acc_den, sem):
    ridxs = [ridx0, ridx1, ridx2, ridx3, ridx4]
    cid = lax.axis_index("c")
    sid = lax.axis_index("s")
    wid = cid * 16 + sid

    pltpu.sync_copy(elr_hbm, elr_t)
    pltpu.sync_copy(mx_hbm, mxbuf)

    zv = jnp.zeros((16,), jnp.float32)
    iota = lax.iota(jnp.int32, 16)

    def zrow(r, carry):
        for q in range(8):
            zbuf[r, pl.ds(q * 16, 16)] = zv
        return carry

    lax.fori_loop(0, 16, zrow, 0)

    def zden(r, carry):
        for q in range(8):
            den_t[r, pl.ds(q * 16, 16)] = zv
        return carry

    lax.fori_loop(0, DR_, zden, 0)
    for r in range(5):
        for q in range(4):
            ridxs[r][pl.ds(q * 16, 16)] = iota + (r * 64 + q * 16)
    @pl.when(sid < 10)
    def _():
        pltpu.sync_copy(zbuf, acc_den.at[pl.ds(sid * 32, 16), :])
        pltpu.sync_copy(zbuf, acc_den.at[pl.ds(sid * 32 + 16, 16), :])

    plsc.subcore_barrier()

    mxv = mxbuf[pl.ds(0, 16)]
    mxs = [mxv[0], mxv[1], mxv[2], mxv[3]]
    ebase = wid * EPT

    def chunk(i, carry):
        off = ebase + i * B_
        pltpu.sync_copy(src_hbm.at[pl.ds(off, B_)], sidx)
        pltpu.sync_copy(dst_hbm.at[pl.ds(off, B_)], didx)
        for g in range(NG_):
            s16 = sidx[pl.ds(g * 16, 16)]
            d16 = didx[pl.ds(g * 16, 16)]
            s8 = s16 * 8
            d8 = d16 * 8
            d4 = d16 * 4
            stg_base = g * 64 + iota * 4
            for hh in range(4):
                elh = plsc.load_gather(elr_t, [s8 + hh])
                erh = plsc.load_gather(elr_t, [d8 + (4 + hh)])
                x = elh + erh
                e = jnp.maximum(x, 0.2 * x)
                y = mxs[hh] + erh
                cc = jnp.maximum(y, 0.2 * y)
                ee = jnp.exp(e - cc)
                plsc.store_scatter(stg, [stg_base + hh], ee)
                idx = d4 + hh
                plsc.addupdate_scatter(den_t, [idx >> 7, idx & 127], ee)
        pltpu.sync_copy(stg, ee_hbm.at[pl.ds(off * 4, 4 * B_)])
        return carry

    lax.fori_loop(0, NCH, chunk, 0)

    for r in range(5):
        pltpu.sync_copy(den_t.at[pl.ds(r * 64, 64), :],
                        acc_den.at[ridxs[r]], add=True)
    plsc.subcore_barrier()

    @pl.when(sid < 10)
    def _():
        pltpu.sync_copy(acc_den.at[pl.ds(sid * 32, 32), :],
                        den_hbm.at[cid, pl.ds(sid * 32, 32), :])


_pass_a = functools.partial(
    pl.kernel,
    mesh=_MESH,
    compiler_params=_SC_PARAMS,
    out_type=[jax.ShapeDtypeStruct((4 * E_,), jnp.float32),
              jax.ShapeDtypeStruct((2, DR_, D_), jnp.float32)],
    scratch_types=[
        pltpu.VMEM((8 * N_,), jnp.float32),
        pltpu.VMEM((DR_, D_), jnp.float32),
        pltpu.VMEM((B_,), jnp.int32),
        pltpu.VMEM((B_,), jnp.int32),
        pltpu.VMEM((4 * B_,), jnp.float32),
        pltpu.VMEM((16,), jnp.float32),
        pltpu.VMEM((16, D_), jnp.float32),
        pltpu.VMEM((64,), jnp.int32),
        pltpu.VMEM((64,), jnp.int32),
        pltpu.VMEM((64,), jnp.int32),
        pltpu.VMEM((64,), jnp.int32),
        pltpu.VMEM((64,), jnp.int32),
        pltpu.VMEM_SHARED((DR_, D_), jnp.float32),
        pltpu.SemaphoreType.DMA,
    ],
)(_pa_body)


def _mid_body(p_ref, o_ref):
    s = p_ref[0] + p_ref[1]
    o_ref[...] = 1.0 / jnp.maximum(s, 1e-30)


_mid = pl.pallas_call(
    _mid_body,
    out_shape=jax.ShapeDtypeStruct((DR_, D_), jnp.float32),
)


HNP = NP_ // 2
EPTB = E_ // 16
NCHB = EPTB // BB_


def _pb_body(src_hbm, dst_hbm, ee_hbm, invd_hbm, h_hbm, out_hbm,
             invd_t, sidx, didx, didx2, stg, hbuf, zbuf, acc, sem):
    cid = lax.axis_index("c")
    sid = lax.axis_index("s")

    pltpu.sync_copy(invd_hbm, invd_t)

    zv = jnp.zeros((16,), jnp.float32)
    iota = lax.iota(jnp.int32, 16)

    def zrow(r, carry):
        for q in range(8):
            zbuf[r, pl.ds(q * 16, 16)] = zv
        return carry

    lax.fori_loop(0, 64, zrow, 0)
    for k in range(5):
        pltpu.sync_copy(zbuf, acc.at[pl.ds(sid * 320 + k * 64, 64), :])

    @pl.when(sid == 15)
    def _():
        pltpu.sync_copy(zbuf.at[pl.ds(0, 8), :], acc.at[pl.ds(HNP, 8), :])

    plsc.subcore_barrier()

    lo = cid * HNP
    ebase = sid * EPTB

    def chunk(i, carry):
        off = ebase + i * BB_
        pltpu.sync_copy(src_hbm.at[pl.ds(off, BB_)], sidx)
        pltpu.sync_copy(dst_hbm.at[pl.ds(off, BB_)], didx)
        pltpu.sync_copy(ee_hbm.at[pl.ds(off * 4, 4 * BB_)], stg)
        pltpu.async_copy(h_hbm.at[sidx], hbuf, sem).wait()
        for g in range(NGB_):
            d16 = didx[pl.ds(g * 16, 16)]
            dl = d16 - lo
            inr = (dl >= 0) & (dl < HNP)
            didx2[pl.ds(g * 16, 16)] = jnp.where(inr, dl, HNP)
            d4 = d16 * 4
            stg_base = g * 64 + iota * 4
            avecs = []
            for hh in range(4):
                eev = plsc.load_gather(stg, [stg_base + hh])
                inv = plsc.load_gather(invd_t, [d4 + hh])
                avecs.append(eev * inv)
            for j in range(16):
                row = g * 16 + j
                for hh in range(4):
                    s = avecs[hh][j]
                    c0 = hh * 32
                    hbuf[row, pl.ds(c0, 16)] = hbuf[row, pl.ds(c0, 16)] * s
                    hbuf[row, pl.ds(c0 + 16, 16)] = (
                        hbuf[row, pl.ds(c0 + 16, 16)] * s)
        pltpu.sync_copy(hbuf, acc.at[didx2], add=True)
        return carry

    lax.fori_loop(0, NCHB, chunk, 0)
    plsc.subcore_barrier()
    for k in range(5):
        r0 = sid * 320 + k * 64
        pltpu.sync_copy(acc.at[pl.ds(r0, 64), :],
                        out_hbm.at[cid, pl.ds(r0, 64), :])


_pass_b = functools.partial(
    pl.kernel,
    mesh=_MESH,
    compiler_params=_SC_PARAMS,
    out_type=jax.ShapeDtypeStruct((2, HNP, D_), jnp.float32),
    scratch_types=[
        pltpu.VMEM((4 * NP_,), jnp.float32),
        pltpu.VMEM((BB_,), jnp.int32),
        pltpu.VMEM((BB_,), jnp.int32),
        pltpu.VMEM((BB_,), jnp.int32),
        pltpu.VMEM((4 * BB_,), jnp.float32),
        pltpu.VMEM((BB_, D_), jnp.float32),
        pltpu.VMEM((64, D_), jnp.float32),
        pltpu.VMEM_SHARED((HNP + 8, D_), jnp.float32),
        pltpu.SemaphoreType.DMA,
    ],
)(_pb_body)


def _fin_body(p_ref, b_ref, o_ref):
    o_ref[...] = p_ref[0] + b_ref[...]


_final = pl.pallas_call(
    _fin_body,
    grid=(NP_ // BN,),
    in_specs=[pl.BlockSpec((1, BN, D_), lambda i: (i // (HNP // BN),
                                                   i % (HNP // BN), 0)),
              pl.BlockSpec((1, D_), lambda i: (0, 0))],
    out_specs=pl.BlockSpec((BN, D_), lambda i: (i, 0)),
    out_shape=jax.ShapeDtypeStruct((NP_, D_), jnp.float32),
)


def kernel(feat, edge_index, W, attn_l, attn_r, bias):
    al = attn_l.reshape(H_ * F_)
    ar = attn_r.reshape(H_ * F_)
    hf = jnp.arange(D_)
    headcol = hf // F_
    M = jnp.zeros((D_, 8), jnp.float32)
    M = M.at[hf, headcol].set(al)
    M = M.at[hf, 4 + headcol].set(ar)

    h_, elr_, mx_ = _prologue(feat, W.T, M)
    src = edge_index[0]
    dst = edge_index[1]
    ee, den_part = _pass_a(src, dst, elr_.reshape(8 * N_),
                           jnp.pad(mx_.reshape(8), (0, 8)))
    invd = _mid(den_part)
    msg_part = _pass_b(src, dst, ee, invd.reshape(4 * NP_), h_)
    out = _final(msg_part, bias.reshape(1, D_))
    return out[:N_].reshape(N_, H_, F_)
